# baseline (speedup 1.0000x reference)
import functools

import jax
import jax.numpy as jnp
from jax import lax
from jax.experimental import pallas as pl
from jax.experimental.pallas import tpu as pltpu
from jax.experimental.pallas import tpu_sc as plsc

N = 10000
E = 320000
D = 128

NC = 2
NS = 16
NW = NC * NS
EW = E // NW
CH = 64
NCH = 160
NCB = 16
NBK = NCH // NCB
EWP = NCH * CH
EPAD = NW * EWP - E
NP = 10112
RT = NP // NS
PAD_ROWS = 112
PAD_LO = NP - PAD_ROWS

_MESH = plsc.VectorSubcoreMesh(core_axis_name="c", subcore_axis_name="s")


NPH = 10240
SEG = NPH // NS


def _deg_body(dsti, out, dst_i, hist, sum_v, tmp_v, acc_s, sem):
    c = lax.axis_index("c")
    s = lax.axis_index("s")
    wid = s * NC + c
    cd = pltpu.async_copy(dsti.at[wid], dst_i, sem)
    zeros16 = jnp.zeros((16,), jnp.float32)

    def zero(i, carry):
        hist[pl.ds(i * 16, 16)] = zeros16
        return carry

    lax.fori_loop(0, NPH // 16, zero, 0)
    cd.wait()
    ones16 = jnp.ones((16,), jnp.float32)

    def chunk(j, carry):
        for k in range(CH // 16):
            iv = dst_i[j, pl.ds(k * 16, 16)]
            plsc.addupdate_scatter(hist, [iv], ones16)
        return carry

    lax.fori_loop(0, NCH, chunk, 0)
    pltpu.sync_copy(hist, acc_s.at[s])
    plsc.subcore_barrier()

    def zero2(i, carry):
        sum_v[pl.ds(i * 16, 16)] = zeros16
        return carry

    lax.fori_loop(0, SEG // 16, zero2, 0)

    def srow(h, carry):
        pltpu.sync_copy(acc_s.at[h, pl.ds(s * SEG, SEG)], tmp_v)
        for k in range(SEG // 16):
            sl = pl.ds(k * 16, 16)
            sum_v[sl] = sum_v[sl] + tmp_v[sl]
        return carry

    lax.fori_loop(0, NS, srow, 0)
    pltpu.sync_copy(sum_v, out.at[c, pl.ds(s * SEG, SEG)])


_deg_call = pl.kernel(
    _deg_body,
    out_type=jax.ShapeDtypeStruct((NC, NPH), jnp.float32),
    mesh=_MESH,
    compiler_params=pltpu.CompilerParams(needs_layout_passes=False),
    scratch_types=[
        pltpu.VMEM((NCH, CH), jnp.int32),
        pltpu.VMEM((NPH,), jnp.float32),
        pltpu.VMEM((SEG,), jnp.float32),
        pltpu.VMEM((SEG,), jnp.float32),
        pltpu.VMEM_SHARED((NS, NPH), jnp.float32),
        pltpu.SemaphoreType.DMA,
    ],
)


def _agg_body(table, srci, dsti, zeros, out,
              src_i, dst_i, ra, rb, rc, rd, acc,
              ga, gb, gc, gd, sa, sb, sc_, sd):
    c = lax.axis_index("c")
    s = lax.axis_index("s")
    wid = s * NC + c
    row_lo = s * RT
    pltpu.sync_copy(zeros.at[pl.ds(row_lo, RT)], acc.at[pl.ds(row_lo, RT)])
    plsc.subcore_barrier()

    bufs = (ra, rb, rc, rd)
    gs = (ga, gb, gc, gd)
    ss = (sa, sb, sc_, sd)

    def step(h, p):
        j = 4 * h + p
        b2 = (p + 2) % 4

        @pl.when(j >= 2)
        def _():
            pltpu.make_async_copy(
                bufs[b2], acc.at[dst_i.at[j - 2]], ss[b2]).wait()

        @pl.when(j + 2 < NCB)
        def _():
            pltpu.async_copy(table.at[src_i.at[j + 2]], bufs[b2], gs[b2])

        pltpu.make_async_copy(table.at[src_i.at[j]], bufs[p], gs[p]).wait()
        pltpu.async_copy(bufs[p], acc.at[dst_i.at[j]], ss[p], add=True)

    def blk(g, carry):
        ci = pltpu.async_copy(srci.at[wid, pl.ds(g * NCB, NCB)], src_i, ga)
        cd = pltpu.async_copy(dsti.at[wid, pl.ds(g * NCB, NCB)], dst_i, gb)
        ci.wait()
        cd.wait()
        pltpu.async_copy(table.at[src_i.at[0]], ra, ga)
        pltpu.async_copy(table.at[src_i.at[1]], rb, gb)

        def quad(h, carry2):
            for p in range(4):
                step(h, p)
            return carry2

        lax.fori_loop(0, NCB // 4, quad, 0)
        pltpu.make_async_copy(bufs[2], acc.at[dst_i.at[NCB - 2]], ss[2]).wait()
        pltpu.make_async_copy(bufs[3], acc.at[dst_i.at[NCB - 1]], ss[3]).wait()
        return carry

    lax.fori_loop(0, NBK, blk, 0)
    plsc.subcore_barrier()
    pltpu.sync_copy(acc.at[pl.ds(row_lo, RT)], out.at[c, pl.ds(row_lo, RT)])


_agg_call = pl.kernel(
    _agg_body,
    out_type=jax.ShapeDtypeStruct((NC, NP, D), jnp.float32),
    mesh=_MESH,
    scratch_types=[
        pltpu.VMEM((NCB, CH), jnp.int32),
        pltpu.VMEM((NCB, CH), jnp.int32),
        pltpu.VMEM((CH, D), jnp.float32),
        pltpu.VMEM((CH, D), jnp.float32),
        pltpu.VMEM((CH, D), jnp.float32),
        pltpu.VMEM((CH, D), jnp.float32),
        pltpu.VMEM_SHARED((NP, D), jnp.float32),
        pltpu.SemaphoreType.DMA,
        pltpu.SemaphoreType.DMA,
        pltpu.SemaphoreType.DMA,
        pltpu.SemaphoreType.DMA,
        pltpu.SemaphoreType.DMA,
        pltpu.SemaphoreType.DMA,
        pltpu.SemaphoreType.DMA,
        pltpu.SemaphoreType.DMA,
    ],
)

RB = 2048
NG = NPH // RB


def _scale1_kernel(x_ref, w_ref, g_ref, y_ref, d_ref):
    deg = g_ref[...]
    dis = jnp.where(deg > 0, lax.rsqrt(deg), 0.0)
    xw = jnp.dot(x_ref[...], w_ref[...], preferred_element_type=jnp.float32)
    y_ref[...] = xw * dis
    d_ref[...] = dis


def _mid_kernel(p_ref, d_ref, b_ref, w_ref, y_ref):
    t = p_ref[0] + p_ref[1]
    dis = d_ref[...]
    h = jnp.maximum(t * dis + b_ref[...], 0.0)
    y_ref[...] = jnp.dot(h, w_ref[...], preferred_element_type=jnp.float32) * dis


def _final_kernel(p_ref, d_ref, b_ref, z_ref):
    z_ref[...] = (p_ref[0] + p_ref[1]) * d_ref[...] + b_ref[...]


def _scale1(x, W1, deg1):
    return pl.pallas_call(
        _scale1_kernel,
        grid=(NG,),
        in_specs=[
            pl.BlockSpec((RB, D), lambda i: (i, 0)),
            pl.BlockSpec((D, D), lambda i: (0, 0)),
            pl.BlockSpec((RB, 1), lambda i: (i, 0)),
        ],
        out_specs=[
            pl.BlockSpec((RB, D), lambda i: (i, 0)),
            pl.BlockSpec((RB, 1), lambda i: (i, 0)),
        ],
        out_shape=[
            jax.ShapeDtypeStruct((N, D), jnp.float32),
            jax.ShapeDtypeStruct((NPH, 1), jnp.float32),
        ],
    )(x, W1, deg1)


def _mid(t1, dis, b1, W2):
    return pl.pallas_call(
        _mid_kernel,
        grid=(NG,),
        in_specs=[
            pl.BlockSpec((NC, RB, D), lambda i: (0, i, 0)),
            pl.BlockSpec((RB, 1), lambda i: (i, 0)),
            pl.BlockSpec((1, D), lambda i: (0, 0)),
            pl.BlockSpec((D, D), lambda i: (0, 0)),
        ],
        out_specs=pl.BlockSpec((RB, D), lambda i: (i, 0)),
        out_shape=jax.ShapeDtypeStruct((N, D), jnp.float32),
    )(t1, dis, b1, W2)


def _final(t2, dis, b2):
    return pl.pallas_call(
        _final_kernel,
        grid=(NG,),
        in_specs=[
            pl.BlockSpec((NC, RB, D), lambda i: (0, i, 0)),
            pl.BlockSpec((RB, 1), lambda i: (i, 0)),
            pl.BlockSpec((1, D), lambda i: (0, 0)),
        ],
        out_specs=pl.BlockSpec((RB, D), lambda i: (i, 0)),
        out_shape=jax.ShapeDtypeStruct((N, D), jnp.float32),
    )(t2, dis, b2)


def kernel(x, edge_index, W1, b1, W2, b2):
    pad = jnp.arange(EPAD, dtype=jnp.int32)
    src = jnp.concatenate(
        [edge_index[0].astype(jnp.int32), pad % N]).reshape(NW, NCH, CH)
    dst = jnp.concatenate(
        [edge_index[1].astype(jnp.int32), PAD_LO + pad % PAD_ROWS]
    ).reshape(NW, NCH, CH)
    zeros128 = jnp.zeros((NP, D), jnp.float32)

    degp = _deg_call(dst)
    deg1 = (degp[0] + degp[1]).reshape(NPH, 1)
    y1, dis = _scale1(x, W1, deg1)
    t1 = _agg_call(y1, src, dst, zeros128)
    y2 = _mid(t1, dis, b1.reshape(1, D), W2)
    t2 = _agg_call(y2, src, dst, zeros128)
    return _final(t2, dis, b2.reshape(1, D))


# speedup vs baseline: 1.0299x; 1.0299x over previous
import functools

import jax
import jax.numpy as jnp
from jax import lax
from jax.experimental import pallas as pl
from jax.experimental.pallas import tpu as pltpu
from jax.experimental.pallas import tpu_sc as plsc

N = 10000
E = 320000
D = 128

NC = 2
NS = 16
NW = NC * NS
EW = E // NW
CH = 128
NCH = 80
NCB = 16
NBK = NCH // NCB
EWP = NCH * CH
EPAD = NW * EWP - E
NP = 10112
RT = NP // NS
PAD_ROWS = 112
PAD_LO = NP - PAD_ROWS

_MESH = plsc.VectorSubcoreMesh(core_axis_name="c", subcore_axis_name="s")


NPH = 10240
SEG = NPH // NS


def _deg_body(dsti, out, dst_i, hist, sum_v, tmp_v, acc_s, sem):
    c = lax.axis_index("c")
    s = lax.axis_index("s")
    wid = s * NC + c
    cd = pltpu.async_copy(dsti.at[wid], dst_i, sem)
    zeros16 = jnp.zeros((16,), jnp.float32)

    def zero(i, carry):
        hist[pl.ds(i * 16, 16)] = zeros16
        return carry

    lax.fori_loop(0, NPH // 16, zero, 0)
    cd.wait()
    ones16 = jnp.ones((16,), jnp.float32)

    def chunk(j, carry):
        for k in range(CH // 16):
            iv = dst_i[j, pl.ds(k * 16, 16)]
            plsc.addupdate_scatter(hist, [iv], ones16)
        return carry

    lax.fori_loop(0, NCH, chunk, 0)
    pltpu.sync_copy(hist, acc_s.at[s])
    plsc.subcore_barrier()

    def zero2(i, carry):
        sum_v[pl.ds(i * 16, 16)] = zeros16
        return carry

    lax.fori_loop(0, SEG // 16, zero2, 0)

    def srow(h, carry):
        pltpu.sync_copy(acc_s.at[h, pl.ds(s * SEG, SEG)], tmp_v)
        for k in range(SEG // 16):
            sl = pl.ds(k * 16, 16)
            sum_v[sl] = sum_v[sl] + tmp_v[sl]
        return carry

    lax.fori_loop(0, NS, srow, 0)
    pltpu.sync_copy(sum_v, out.at[c, pl.ds(s * SEG, SEG)])


_deg_call = pl.kernel(
    _deg_body,
    out_type=jax.ShapeDtypeStruct((NC, NPH), jnp.float32),
    mesh=_MESH,
    compiler_params=pltpu.CompilerParams(needs_layout_passes=False),
    scratch_types=[
        pltpu.VMEM((NCH, CH), jnp.int32),
        pltpu.VMEM((NPH,), jnp.float32),
        pltpu.VMEM((SEG,), jnp.float32),
        pltpu.VMEM((SEG,), jnp.float32),
        pltpu.VMEM_SHARED((NS, NPH), jnp.float32),
        pltpu.SemaphoreType.DMA,
    ],
)


def _agg_body(table, srci, dsti, zeros, out,
              src_i, dst_i, rows_a, rows_b, acc,
              sem_a, sem_b, sem_sa, sem_sb):
    c = lax.axis_index("c")
    s = lax.axis_index("s")
    wid = s * NC + c
    row_lo = s * RT
    pltpu.sync_copy(zeros.at[pl.ds(row_lo, RT)], acc.at[pl.ds(row_lo, RT)])
    plsc.subcore_barrier()

    def step(j, buf, gsem, ssem, obuf, ogsem, ossem):
        @pl.when(j >= 1)
        def _():
            pltpu.make_async_copy(obuf, acc.at[dst_i.at[j - 1]], ossem).wait()

        @pl.when(j + 1 < NCB)
        def _():
            pltpu.async_copy(table.at[src_i.at[j + 1]], obuf, ogsem)

        pltpu.make_async_copy(table.at[src_i.at[j]], buf, gsem).wait()
        pltpu.async_copy(buf, acc.at[dst_i.at[j]], ssem, add=True)

    def blk(g, carry):
        ci = pltpu.async_copy(srci.at[wid, pl.ds(g * NCB, NCB)], src_i, sem_a)
        cd = pltpu.async_copy(dsti.at[wid, pl.ds(g * NCB, NCB)], dst_i, sem_b)
        ci.wait()
        cd.wait()
        pltpu.async_copy(table.at[src_i.at[0]], rows_a, sem_a)

        def pair(h, carry2):
            step(2 * h, rows_a, sem_a, sem_sa, rows_b, sem_b, sem_sb)
            step(2 * h + 1, rows_b, sem_b, sem_sb, rows_a, sem_a, sem_sa)
            return carry2

        lax.fori_loop(0, NCB // 2, pair, 0)
        pltpu.make_async_copy(
            rows_b, acc.at[dst_i.at[NCB - 1]], sem_sb).wait()
        return carry

    lax.fori_loop(0, NBK, blk, 0)
    plsc.subcore_barrier()
    pltpu.sync_copy(acc.at[pl.ds(row_lo, RT)], out.at[c, pl.ds(row_lo, RT)])


_agg_call = pl.kernel(
    _agg_body,
    out_type=jax.ShapeDtypeStruct((NC, NP, D), jnp.float32),
    mesh=_MESH,
    scratch_types=[
        pltpu.VMEM((NCB, CH), jnp.int32),
        pltpu.VMEM((NCB, CH), jnp.int32),
        pltpu.VMEM((CH, D), jnp.float32),
        pltpu.VMEM((CH, D), jnp.float32),
        pltpu.VMEM_SHARED((NP, D), jnp.float32),
        pltpu.SemaphoreType.DMA,
        pltpu.SemaphoreType.DMA,
        pltpu.SemaphoreType.DMA,
        pltpu.SemaphoreType.DMA,
    ],
)

RB = 2048
NG = NPH // RB


def _scale1_kernel(x_ref, w_ref, g_ref, y_ref, d_ref):
    deg = g_ref[...]
    dis = jnp.where(deg > 0, lax.rsqrt(deg), 0.0)
    xw = jnp.dot(x_ref[...], w_ref[...], preferred_element_type=jnp.float32)
    y_ref[...] = xw * dis
    d_ref[...] = dis


def _mid_kernel(p_ref, d_ref, b_ref, w_ref, y_ref):
    t = p_ref[0] + p_ref[1]
    dis = d_ref[...]
    h = jnp.maximum(t * dis + b_ref[...], 0.0)
    y_ref[...] = jnp.dot(h, w_ref[...], preferred_element_type=jnp.float32) * dis


def _final_kernel(p_ref, d_ref, b_ref, z_ref):
    z_ref[...] = (p_ref[0] + p_ref[1]) * d_ref[...] + b_ref[...]


def _scale1(x, W1, deg1):
    return pl.pallas_call(
        _scale1_kernel,
        grid=(NG,),
        in_specs=[
            pl.BlockSpec((RB, D), lambda i: (i, 0)),
            pl.BlockSpec((D, D), lambda i: (0, 0)),
            pl.BlockSpec((RB, 1), lambda i: (i, 0)),
        ],
        out_specs=[
            pl.BlockSpec((RB, D), lambda i: (i, 0)),
            pl.BlockSpec((RB, 1), lambda i: (i, 0)),
        ],
        out_shape=[
            jax.ShapeDtypeStruct((N, D), jnp.float32),
            jax.ShapeDtypeStruct((NPH, 1), jnp.float32),
        ],
    )(x, W1, deg1)


def _mid(t1, dis, b1, W2):
    return pl.pallas_call(
        _mid_kernel,
        grid=(NG,),
        in_specs=[
            pl.BlockSpec((NC, RB, D), lambda i: (0, i, 0)),
            pl.BlockSpec((RB, 1), lambda i: (i, 0)),
            pl.BlockSpec((1, D), lambda i: (0, 0)),
            pl.BlockSpec((D, D), lambda i: (0, 0)),
        ],
        out_specs=pl.BlockSpec((RB, D), lambda i: (i, 0)),
        out_shape=jax.ShapeDtypeStruct((N, D), jnp.float32),
    )(t1, dis, b1, W2)


def _final(t2, dis, b2):
    return pl.pallas_call(
        _final_kernel,
        grid=(NG,),
        in_specs=[
            pl.BlockSpec((NC, RB, D), lambda i: (0, i, 0)),
            pl.BlockSpec((RB, 1), lambda i: (i, 0)),
            pl.BlockSpec((1, D), lambda i: (0, 0)),
        ],
        out_specs=pl.BlockSpec((RB, D), lambda i: (i, 0)),
        out_shape=jax.ShapeDtypeStruct((N, D), jnp.float32),
    )(t2, dis, b2)


def kernel(x, edge_index, W1, b1, W2, b2):
    pad = jnp.arange(EPAD, dtype=jnp.int32)
    src = jnp.concatenate(
        [edge_index[0].astype(jnp.int32), pad % N]).reshape(NW, NCH, CH)
    dst = jnp.concatenate(
        [edge_index[1].astype(jnp.int32), PAD_LO + pad % PAD_ROWS]
    ).reshape(NW, NCH, CH)
    zeros128 = jnp.zeros((NP, D), jnp.float32)

    degp = _deg_call(dst)
    deg1 = (degp[0] + degp[1]).reshape(NPH, 1)
    y1, dis = _scale1(x, W1, deg1)
    t1 = _agg_call(y1, src, dst, zeros128)
    y2 = _mid(t1, dis, b1.reshape(1, D), W2)
    t2 = _agg_call(y2, src, dst, zeros128)
    return _final(t2, dis, b2.reshape(1, D))


# speedup vs baseline: 1.0306x; 1.0007x over previous
import jax
import jax.numpy as jnp
from jax import lax
from jax.experimental import pallas as pl
from jax.experimental.pallas import tpu as pltpu
from jax.experimental.pallas import tpu_sc as plsc

N = 10000
E = 320000
D = 128

NC = 2
NS = 16
NW = NC * NS
EW = E // NW
CH = 128
NCH = 80
NCB = 16
NBK = NCH // NCB
EWP = NCH * CH
EPAD = NW * EWP - E
NP = 10112
RT = NP // NS
PAD_ROWS = 112
PAD_LO = NP - PAD_ROWS

_MESH = plsc.VectorSubcoreMesh(core_axis_name="c", subcore_axis_name="s")


NPH = 10240
SEG = NPH // NS


def _deg_body(dsti, out, dst_i, hist, sum_v, tmp_v, acc_s, sem):
    c = lax.axis_index("c")
    s = lax.axis_index("s")
    wid = s * NC + c
    cd = pltpu.async_copy(dsti.at[wid], dst_i, sem)
    zeros16 = jnp.zeros((16,), jnp.float32)

    def zero(i, carry):
        hist[pl.ds(i * 16, 16)] = zeros16
        return carry

    lax.fori_loop(0, NPH // 16, zero, 0)
    cd.wait()
    ones16 = jnp.ones((16,), jnp.float32)

    def chunk(j, carry):
        for k in range(CH // 16):
            iv = dst_i[j, pl.ds(k * 16, 16)]
            plsc.addupdate_scatter(hist, [iv], ones16)
        return carry

    lax.fori_loop(0, NCH, chunk, 0)
    pltpu.sync_copy(hist, acc_s.at[s])
    plsc.subcore_barrier()

    def zero2(i, carry):
        sum_v[pl.ds(i * 16, 16)] = zeros16
        return carry

    lax.fori_loop(0, SEG // 16, zero2, 0)

    def srow(h, carry):
        pltpu.sync_copy(acc_s.at[h, pl.ds(s * SEG, SEG)], tmp_v)
        for k in range(SEG // 16):
            sl = pl.ds(k * 16, 16)
            sum_v[sl] = sum_v[sl] + tmp_v[sl]
        return carry

    lax.fori_loop(0, NS, srow, 0)
    pltpu.sync_copy(sum_v, out.at[c, pl.ds(s * SEG, SEG)])


_deg_call = pl.kernel(
    _deg_body,
    out_type=jax.ShapeDtypeStruct((NC, NPH), jnp.float32),
    mesh=_MESH,
    compiler_params=pltpu.CompilerParams(needs_layout_passes=False),
    scratch_types=[
        pltpu.VMEM((NCH, CH), jnp.int32),
        pltpu.VMEM((NPH,), jnp.float32),
        pltpu.VMEM((SEG,), jnp.float32),
        pltpu.VMEM((SEG,), jnp.float32),
        pltpu.VMEM_SHARED((NS, NPH), jnp.float32),
        pltpu.SemaphoreType.DMA,
    ],
)


def _agg_body(table, srci, dsti, zeros, out,
              src_i, dst_i, rows_a, rows_b, acc,
              sem_a, sem_b, sem_sa, sem_sb):
    c = lax.axis_index("c")
    s = lax.axis_index("s")
    wid = s * NC + c
    row_lo = s * RT
    pltpu.sync_copy(zeros.at[pl.ds(row_lo, RT)], acc.at[pl.ds(row_lo, RT)])
    plsc.subcore_barrier()

    def step(j, buf, gsem, ssem, obuf, ogsem, ossem):
        @pl.when(j >= 1)
        def _():
            pltpu.make_async_copy(obuf, acc.at[dst_i.at[j - 1]], ossem).wait()

        @pl.when(j + 1 < NCB)
        def _():
            pltpu.async_copy(table.at[src_i.at[j + 1]], obuf, ogsem)

        pltpu.make_async_copy(table.at[src_i.at[j]], buf, gsem).wait()
        pltpu.async_copy(buf, acc.at[dst_i.at[j]], ssem, add=True)

    def blk(g, carry):
        ci = pltpu.async_copy(srci.at[wid, pl.ds(g * NCB, NCB)], src_i, sem_a)
        cd = pltpu.async_copy(dsti.at[wid, pl.ds(g * NCB, NCB)], dst_i, sem_b)
        ci.wait()
        cd.wait()
        pltpu.async_copy(table.at[src_i.at[0]], rows_a, sem_a)

        def pair(h, carry2):
            step(2 * h, rows_a, sem_a, sem_sa, rows_b, sem_b, sem_sb)
            step(2 * h + 1, rows_b, sem_b, sem_sb, rows_a, sem_a, sem_sa)
            return carry2

        lax.fori_loop(0, NCB // 2, pair, 0)
        pltpu.make_async_copy(
            rows_b, acc.at[dst_i.at[NCB - 1]], sem_sb).wait()
        return carry

    lax.fori_loop(0, NBK, blk, 0)
    plsc.subcore_barrier()
    pltpu.sync_copy(acc.at[pl.ds(row_lo, RT)], out.at[c, pl.ds(row_lo, RT)])


_agg_call = pl.kernel(
    _agg_body,
    out_type=jax.ShapeDtypeStruct((NC, NP, D), jnp.float32),
    mesh=_MESH,
    scratch_types=[
        pltpu.VMEM((NCB, CH), jnp.int32),
        pltpu.VMEM((NCB, CH), jnp.int32),
        pltpu.VMEM((CH, D), jnp.float32),
        pltpu.VMEM((CH, D), jnp.float32),
        pltpu.VMEM_SHARED((NP, D), jnp.float32),
        pltpu.SemaphoreType.DMA,
        pltpu.SemaphoreType.DMA,
        pltpu.SemaphoreType.DMA,
        pltpu.SemaphoreType.DMA,
    ],
)

RB = 2048
NG = NPH // RB


def _scale1_kernel(x_ref, w_ref, g_ref, y_ref, d_ref):
    deg = g_ref[...]
    dis = jnp.where(deg > 0, lax.rsqrt(deg), 0.0)
    xw = jnp.dot(x_ref[...], w_ref[...], preferred_element_type=jnp.float32)
    y_ref[...] = xw * dis
    d_ref[...] = dis


def _mid_kernel(p_ref, d_ref, b_ref, w_ref, y_ref):
    t = p_ref[0] + p_ref[1]
    dis = d_ref[...]
    h = jnp.maximum(t * dis + b_ref[...], 0.0)
    y_ref[...] = jnp.dot(h, w_ref[...], preferred_element_type=jnp.float32) * dis


def _final_kernel(p_ref, d_ref, b_ref, z_ref):
    z_ref[...] = (p_ref[0] + p_ref[1]) * d_ref[...] + b_ref[...]


def _scale1(x, W1, deg1):
    return pl.pallas_call(
        _scale1_kernel,
        grid=(NG,),
        in_specs=[
            pl.BlockSpec((RB, D), lambda i: (i, 0)),
            pl.BlockSpec((D, D), lambda i: (0, 0)),
            pl.BlockSpec((RB, 1), lambda i: (i, 0)),
        ],
        out_specs=[
            pl.BlockSpec((RB, D), lambda i: (i, 0)),
            pl.BlockSpec((RB, 1), lambda i: (i, 0)),
        ],
        out_shape=[
            jax.ShapeDtypeStruct((N, D), jnp.float32),
            jax.ShapeDtypeStruct((NPH, 1), jnp.float32),
        ],
    )(x, W1, deg1)


def _mid(t1, dis, b1, W2):
    return pl.pallas_call(
        _mid_kernel,
        grid=(NG,),
        in_specs=[
            pl.BlockSpec((NC, RB, D), lambda i: (0, i, 0)),
            pl.BlockSpec((RB, 1), lambda i: (i, 0)),
            pl.BlockSpec((1, D), lambda i: (0, 0)),
            pl.BlockSpec((D, D), lambda i: (0, 0)),
        ],
        out_specs=pl.BlockSpec((RB, D), lambda i: (i, 0)),
        out_shape=jax.ShapeDtypeStruct((N, D), jnp.float32),
    )(t1, dis, b1, W2)


def _final(t2, dis, b2):
    return pl.pallas_call(
        _final_kernel,
        grid=(NG,),
        in_specs=[
            pl.BlockSpec((NC, RB, D), lambda i: (0, i, 0)),
            pl.BlockSpec((RB, 1), lambda i: (i, 0)),
            pl.BlockSpec((1, D), lambda i: (0, 0)),
        ],
        out_specs=pl.BlockSpec((RB, D), lambda i: (i, 0)),
        out_shape=jax.ShapeDtypeStruct((N, D), jnp.float32),
    )(t2, dis, b2)


def kernel(x, edge_index, W1, b1, W2, b2):
    pad = jnp.arange(EPAD, dtype=jnp.int32)
    src = jnp.concatenate(
        [edge_index[0].astype(jnp.int32), pad % N]).reshape(NW, NCH, CH)
    dst = jnp.concatenate(
        [edge_index[1].astype(jnp.int32), PAD_LO + pad % PAD_ROWS]
    ).reshape(NW, NCH, CH)
    zeros128 = jnp.zeros((NP, D), jnp.float32)

    degp = _deg_call(dst)
    deg1 = (degp[0] + degp[1]).reshape(NPH, 1)
    y1, dis = _scale1(x, W1, deg1)
    t1 = _agg_call(y1, src, dst, zeros128)
    y2 = _mid(t1, dis, b1.reshape(1, D), W2)
    t2 = _agg_call(y2, src, dst, zeros128)
    return _final(t2, dis, b2.reshape(1, D))
